# Initial kernel scaffold; baseline (speedup 1.0000x reference)
import jax
import jax.numpy as jnp
from jax.experimental import pallas as pl


def kernel(input):
    raise NotImplementedError("write your pallas kernel here")


# speedup vs baseline: 4.4665x; 4.4665x over previous
import functools

import jax
import jax.numpy as jnp
from jax.experimental import pallas as pl

K = 20
N = 1024
ROWS = 32 * 768
BLOCK_R = 256

def _body(x_ref, o_ref):
    _SIGN = jnp.int32(-(2**31))
    _MAXP = jnp.int32(0x7FFFFFFF)
    x = x_ref[...]
    r = x.shape[0]
    i = jax.lax.bitcast_convert_type(x, jnp.int32)
    skey = i ^ (jax.lax.shift_right_arithmetic(i, 31) & _MAXP)

    kf = jnp.float32(K)
    p_hi = jnp.zeros((r, 1), jnp.int32)
    p_lo = jnp.zeros((r, 1), jnp.int32)
    for b in range(31, -1, -1):
        bit = jnp.int32(1 << b) if b < 31 else _SIGN
        cand_hi = p_hi | bit
        cand_lo = p_lo | bit
        c_hi = jnp.sum((skey >= (cand_hi ^ _SIGN)).astype(jnp.float32),
                       axis=1, keepdims=True)
        c_lo = jnp.sum((skey <= (~cand_lo ^ _SIGN)).astype(jnp.float32),
                       axis=1, keepdims=True)
        p_hi = jnp.where(c_hi >= kf, cand_hi, p_hi)
        p_lo = jnp.where(c_lo >= kf, cand_lo, p_lo)

    ts_hi = p_hi ^ _SIGN
    ts_lo = ~p_lo ^ _SIGN

    gt = skey > ts_hi
    lt = skey < ts_lo
    cnt_gt = jnp.sum(gt.astype(jnp.float32), axis=1, keepdims=True)
    cnt_lt = jnp.sum(lt.astype(jnp.float32), axis=1, keepdims=True)
    sum_gt = jnp.sum(jnp.where(gt, x, 0.0), axis=1, keepdims=True)
    sum_lt = jnp.sum(jnp.where(lt, x, 0.0), axis=1, keepdims=True)

    iv_hi = jnp.where(ts_hi >= 0, ts_hi, ts_hi ^ _MAXP)
    iv_lo = jnp.where(ts_lo >= 0, ts_lo, ts_lo ^ _MAXP)
    v_hi = jax.lax.bitcast_convert_type(iv_hi, jnp.float32)
    v_lo = jax.lax.bitcast_convert_type(iv_lo, jnp.float32)

    top = sum_gt + v_hi * (kf - cnt_gt)
    bot = sum_lt + v_lo * (kf - cnt_lt)
    o_ref[...] = ((top + bot) * jnp.float32(0.5 / K))[:, 0]


def kernel(input):
    bsz, nch, h, w = input.shape
    flat = input.reshape(bsz * nch, h * w)
    out = pl.pallas_call(
        _body,
        grid=(ROWS // BLOCK_R,),
        in_specs=[pl.BlockSpec((BLOCK_R, N), lambda b: (b, 0))],
        out_specs=pl.BlockSpec((BLOCK_R,), lambda b: (b,)),
        out_shape=jax.ShapeDtypeStruct((ROWS,), jnp.float32),
    )(flat)
    return out.reshape(bsz, nch)
